# baseline (speedup 1.0000x reference)
import functools

import jax
import jax.numpy as jnp
from jax import lax
from jax.experimental import pallas as pl
from jax.experimental.pallas import tpu as pltpu
from jax.experimental.pallas import tpu_sc as plsc

_N = 10000
_D = 128
_NC, _NS = 2, 16
_NW = _NC * _NS
_NPT = 320
_NPAD = _NW * _NPT
_CH = 128
_EPT_CH = 80
_EPAD = _NW * _EPT_CH * _CH
_SCHK = 8
_RPS = _NPAD // _NS

_AGG_MODES = ['sum', 'mean', 'max', 'sum', 'mean', 'max', 'sum', 'mean',
              'max', 'sum', 'max', 'mean']

_SC_PARAMS = pltpu.CompilerParams(needs_layout_passes=False)
_MESH = dict(core_axis_name="c", subcore_axis_name="s",
             num_cores=_NC, num_subcores=_NS)


def _extract(v16, k):
    return jnp.sum(jnp.where(lax.iota(jnp.int32, 16) == k, v16, 0))


def _make_stream_sum_kernel():
    @functools.partial(
        pl.kernel,
        out_type=jax.ShapeDtypeStruct((2 * _NPAD, _D), jnp.float32),
        mesh=plsc.VectorSubcoreMesh(**_MESH),
        compiler_params=_SC_PARAMS,
        scratch_types=[
            pltpu.VMEM((_EPT_CH // 2, _CH), jnp.int32),
            pltpu.VMEM((_EPT_CH // 2, _CH), jnp.int32),
            pltpu.VMEM((_CH, _D), jnp.float32),
            pltpu.VMEM((_CH, _D), jnp.float32),
            pltpu.VMEM_SHARED((_NPAD, _D), jnp.float32),
            pltpu.SemaphoreType.DMA,
            pltpu.SemaphoreType.DMA,
        ],
    )
    def k(h_hbm, src_hbm, dst_hbm, out_hbm,
          sv, dv, mv0, mv1, aggs, sem0, sem1):
        cid = lax.axis_index("c")
        sid = lax.axis_index("s")
        wid = sid * _NC + cid
        half_ch = _EPT_CH // 2

        def zrow(r, c):
            for kk in range(_D // 16):
                mv0[r, pl.ds(kk * 16, 16)] = jnp.zeros((16,), jnp.float32)
            return c
        lax.fori_loop(0, _CH, zrow, 0)
        for kk in range(_RPS // _CH):
            pltpu.sync_copy(mv0, aggs.at[pl.ds(sid * _RPS + kk * _CH, _CH)])
        plsc.subcore_barrier()

        for half in range(2):
            row0 = wid * _EPT_CH + half * half_ch
            pltpu.sync_copy(src_hbm.at[pl.ds(row0, half_ch)], sv)
            pltpu.sync_copy(dst_hbm.at[pl.ds(row0, half_ch)], dv)

            pltpu.async_copy(h_hbm.at[sv.at[0]], mv0, sem0)

            def body(i, c):
                c0 = 2 * i
                c1 = c0 + 1
                c2 = c0 + 2
                pltpu.async_copy(h_hbm.at[sv.at[c1]], mv1, sem1)
                pltpu.make_async_copy(h_hbm.at[sv.at[c0]], mv0, sem0).wait()
                pltpu.sync_copy(mv0, aggs.at[dv.at[c0]], add=True)

                @pl.when(c2 < half_ch)
                def _():
                    pltpu.async_copy(h_hbm.at[sv.at[c2]], mv0, sem0)

                pltpu.make_async_copy(h_hbm.at[sv.at[c1]], mv1, sem1).wait()
                pltpu.sync_copy(mv1, aggs.at[dv.at[c1]], add=True)
                return c
            lax.fori_loop(0, half_ch // 2, body, 0)

        plsc.subcore_barrier()
        for kk in range(_RPS // _CH):
            r = sid * _RPS + kk * _CH
            pltpu.sync_copy(aggs.at[pl.ds(r, _CH)],
                            out_hbm.at[pl.ds(cid * _NPAD + r, _CH)])

    return k


_OPS = {'sum': jnp.add, 'mean': jnp.add, 'max': jnp.maximum}


def _make_csr_kernel(op):
    ident = 0.0 if op in ('sum', 'mean') else float('-inf')

    @functools.partial(
        pl.kernel,
        out_type=jax.ShapeDtypeStruct((_NPAD * _D,), jnp.float32),
        mesh=plsc.VectorSubcoreMesh(**_MESH),
        compiler_params=_SC_PARAMS,
        scratch_types=[
            pltpu.VMEM((48,), jnp.int32),
            pltpu.VMEM((328,), jnp.int32),
            pltpu.VMEM((_SCHK, _CH), jnp.int32),
            pltpu.VMEM((16,), jnp.int32),
            pltpu.VMEM((16,), jnp.int32),
            pltpu.VMEM((_CH, _D), jnp.float32),
            pltpu.VMEM((_CH, _D), jnp.float32),
            pltpu.VMEM((_NPT * _D,), jnp.float32),
            pltpu.SemaphoreType.DMA,
            pltpu.SemaphoreType.DMA,
        ],
    )
    def k(h_hbm, src_hbm, rp_hbm, cn0_hbm, cn1_hbm, bounds_hbm, out_hbm,
          bv, rpv, sv, c0v, c1v, mv0, mv1, av, sem0, sem1):
        cid = lax.axis_index("c")
        sid = lax.axis_index("s")
        wid = sid * _NC + cid
        node_base = wid * _NPT

        def zbody(i, c):
            av[pl.ds(i * 16, 16)] = jnp.zeros((16,), jnp.float32)
            return c
        lax.fori_loop(0, _NPT * _D // 16, zbody, 0)

        pltpu.sync_copy(bounds_hbm, bv)
        pltpu.sync_copy(rp_hbm.at[pl.ds(node_base, 328)], rpv)
        b0 = bv[pl.ds(0, 16)]
        b1 = bv[pl.ds(16, 16)]
        b2 = bv[pl.ds(32, 16)]

        def bval(t):
            return _extract(b0, t) + _extract(b1, t - 16) + _extract(b2, t - 32)

        def rpval(nl):
            v16 = rpv[pl.ds(pl.multiple_of((nl // 16) * 16, 16), 16)]
            return _extract(v16, nl & 15)

        e_start = bval(wid)
        e_end = bval(wid + 1)
        sch = _SCHK * _CH
        s_lo = (e_start // sch) * sch
        n_sch = (e_end - s_lo + sch - 1) // sch

        mvs = (mv0, mv1)
        sems = (sem0, sem1)

        def sch_body(s, acc):
            row0 = pl.multiple_of((s_lo // _CH) + s * _SCHK, _SCHK)
            pltpu.sync_copy(src_hbm.at[pl.ds(row0, _SCHK)], sv)
            pltpu.sync_copy(cn0_hbm.at[pl.ds(row0, 16)], c0v)
            pltpu.sync_copy(cn1_hbm.at[pl.ds(row0, 16)], c1v)
            cn0_16 = c0v[...]
            cn1_16 = c1v[...]
            pltpu.async_copy(h_hbm.at[sv.at[0]], mv0, sem0)
            for kk in range(_SCHK):
                mk, sk = mvs[kk % 2], sems[kk % 2]
                if kk + 1 < _SCHK:
                    pltpu.async_copy(h_hbm.at[sv.at[kk + 1]],
                                     mvs[(kk + 1) % 2], sems[(kk + 1) % 2])
                pltpu.make_async_copy(h_hbm.at[sv.at[kk]], mk, sk).wait()
                ch_lo = s_lo + s * sch + kk * _CH
                ch_hi = ch_lo + _CH
                nlo = jnp.maximum(_extract(cn0_16, kk), node_base)
                nhi = jnp.minimum(_extract(cn1_16, kk), node_base + _NPT - 1)

                def node_body(n, acc2, _mk=mk, _lo=ch_lo, _hi=ch_hi):
                    nl = n - node_base
                    rpn = rpval(nl)
                    rpn1 = rpval(nl + 1)
                    e0 = jnp.maximum(rpn, _lo)
                    e1 = jnp.minimum(rpn1, _hi)

                    def edge_body(e, acc3):
                        el = e - _lo
                        return tuple(
                            _OPS[op](acc3[cc], _mk[el, pl.ds(cc * 16, 16)])
                            for cc in range(8))
                    acc2 = lax.fori_loop(e0, e1, edge_body, acc2)
                    done = jnp.logical_and(rpn1 <= _hi, rpn1 > rpn)

                    @pl.when(done)
                    def _():
                        if op == 'mean':
                            degv = jnp.full((16,), rpn1 - rpn,
                                            jnp.int32).astype(jnp.float32)
                            sc = 1.0 / degv
                        else:
                            sc = 1.0
                        for cc in range(8):
                            av[pl.ds(nl * _D + cc * 16, 16)] = acc2[cc] * sc
                    return tuple(
                        jnp.where(done, ident, acc2[cc]).astype(jnp.float32)
                        for cc in range(8))
                acc = lax.fori_loop(nlo, nhi + 1, node_body, acc)
            return acc

        acc0 = tuple(jnp.full((16,), ident, jnp.float32) for _ in range(8))
        lax.fori_loop(0, n_sch, sch_body, acc0)
        pltpu.sync_copy(av.at[pl.ds(0, _NPT * _D)],
                        out_hbm.at[pl.ds(node_base * _D, _NPT * _D)])

    return k


_agg_stream_sum = _make_stream_sum_kernel()
_agg_csr = {m: _make_csr_kernel(m) for m in ('sum', 'mean', 'max')}

_BR = 1024


def _sage2_body(p0_ref, p1_ref, sc_ref, h_ref, wl_ref, b_ref, wr_ref, o_ref):
    a = (p0_ref[...] + p1_ref[...]) * sc_ref[...]
    y = jnp.dot(a, wl_ref[...], preferred_element_type=jnp.float32)
    y = y + jnp.dot(h_ref[...], wr_ref[...], preferred_element_type=jnp.float32)
    o_ref[...] = jnp.maximum(y + b_ref[...], 0.0)


def _sage1_body(agg_ref, h_ref, wl_ref, b_ref, wr_ref, o_ref):
    y = jnp.dot(agg_ref[...], wl_ref[...], preferred_element_type=jnp.float32)
    y = y + jnp.dot(h_ref[...], wr_ref[...], preferred_element_type=jnp.float32)
    o_ref[...] = jnp.maximum(y + b_ref[...], 0.0)


def _final_body(p0_ref, p1_ref, sc_ref, h_ref, wl_ref, b_ref, wr_ref,
                w1_ref, b1_ref, w2_ref, b2_ref, o_ref):
    a = (p0_ref[...] + p1_ref[...]) * sc_ref[...]
    y = jnp.dot(a, wl_ref[...], preferred_element_type=jnp.float32)
    y = y + jnp.dot(h_ref[...], wr_ref[...], preferred_element_type=jnp.float32)
    y = jnp.maximum(y + b_ref[...], 0.0)
    y = jnp.maximum(
        jnp.dot(y, w1_ref[...], preferred_element_type=jnp.float32)
        + b1_ref[...], 0.0)
    o_ref[...] = (jnp.dot(y, w2_ref[...], preferred_element_type=jnp.float32)
                  + b2_ref[...])


def _final1_body(agg_ref, h_ref, wl_ref, b_ref, wr_ref,
                 w1_ref, b1_ref, w2_ref, b2_ref, o_ref):
    y = jnp.dot(agg_ref[...], wl_ref[...], preferred_element_type=jnp.float32)
    y = y + jnp.dot(h_ref[...], wr_ref[...], preferred_element_type=jnp.float32)
    y = jnp.maximum(y + b_ref[...], 0.0)
    y = jnp.maximum(
        jnp.dot(y, w1_ref[...], preferred_element_type=jnp.float32)
        + b1_ref[...], 0.0)
    o_ref[...] = (jnp.dot(y, w2_ref[...], preferred_element_type=jnp.float32)
                  + b2_ref[...])


def _row_spec():
    return pl.BlockSpec((_BR, _D), lambda i: (i, 0))


def _scale_spec():
    return pl.BlockSpec((_BR, 1), lambda i: (i, 0))


def _full_spec(shape):
    return pl.BlockSpec(shape, lambda i: (0, 0))


def _tc_call(body, specs, args):
    return pl.pallas_call(
        body,
        grid=(_NPAD // _BR,),
        in_specs=specs,
        out_specs=_row_spec(),
        out_shape=jax.ShapeDtypeStruct((_NPAD, _D), jnp.float32),
    )(*args)


def _tc_sage2(p0, p1, scale, h, wlT, bl, wrT):
    specs = [_row_spec(), _row_spec(), _scale_spec(), _row_spec(),
             _full_spec((_D, _D)), _full_spec((1, _D)), _full_spec((_D, _D))]
    return _tc_call(_sage2_body, specs, (p0, p1, scale, h, wlT, bl, wrT))


def _tc_sage1(agg, h, wlT, bl, wrT):
    specs = [_row_spec(), _row_spec(), _full_spec((_D, _D)),
             _full_spec((1, _D)), _full_spec((_D, _D))]
    return _tc_call(_sage1_body, specs, (agg, h, wlT, bl, wrT))


def _tc_final1(agg, h, wlT, bl, wrT, w1T, b1, w2T, b2):
    specs = [_row_spec(), _row_spec(), _full_spec((_D, _D)),
             _full_spec((1, _D)), _full_spec((_D, _D)), _full_spec((_D, _D)),
             _full_spec((1, _D)), _full_spec((_D, _D)), _full_spec((1, _D))]
    return _tc_call(_final1_body, specs, (agg, h, wlT, bl, wrT, w1T, b1, w2T, b2))


def _tc_final(p0, p1, scale, h, wlT, bl, wrT, w1T, b1, w2T, b2):
    specs = [_row_spec(), _row_spec(), _scale_spec(), _row_spec(),
             _full_spec((_D, _D)), _full_spec((1, _D)), _full_spec((_D, _D)),
             _full_spec((_D, _D)), _full_spec((1, _D)),
             _full_spec((_D, _D)), _full_spec((1, _D))]
    return _tc_call(_final_body, specs,
                    (p0, p1, scale, h, wlT, bl, wrT, w1T, b1, w2T, b2))


def kernel(x, edge_index, params):
    src = edge_index[0].astype(jnp.int32)
    dst = edge_index[1].astype(jnp.int32)
    n_e = src.shape[0]
    n_rows = n_e // _CH

    dst_s, src_s = dst, src
    src_s2 = jnp.concatenate(
        [src_s.reshape(n_rows, _CH),
         jnp.zeros((4, _CH), jnp.int32)])
    dst_s2 = dst_s.reshape(n_rows, _CH)
    cn_pad = jnp.full((12,), _NPAD - 1, jnp.int32)
    cn0 = jnp.concatenate([dst_s2[:, 0], cn_pad])
    cn1 = jnp.concatenate([dst_s2[:, -1], cn_pad])

    tile_nodes = jnp.arange(0, _NPAD + 1, _NPT, dtype=jnp.int32)
    bounds = jnp.searchsorted(dst_s, tile_nodes).astype(jnp.int32)
    bounds_pad = jnp.zeros((48,), jnp.int32).at[:_NW + 1].set(bounds)

    rowptr = jnp.searchsorted(
        dst_s, jnp.arange(_NPAD + 1, dtype=jnp.int32)).astype(jnp.int32)
    rp_pad = jnp.concatenate([rowptr, jnp.full((7,), n_e, jnp.int32)])

    h = jnp.zeros((_NPAD, _D), jnp.float32).at[:_N].set(x)

    for i, aggr in enumerate(_AGG_MODES):
        wlT = params['Wl%d' % i].T
        bl = params['bl%d' % i].reshape(1, _D)
        wrT = params['Wr%d' % i].T
        agg = _agg_csr[aggr](h, src_s2, rp_pad, cn0, cn1,
                             bounds_pad).reshape(_NPAD, _D)
        if i < len(_AGG_MODES) - 1:
            h = _tc_sage1(agg, h, wlT, bl, wrT)
        else:
            out = _tc_final1(agg, h, wlT, bl, wrT,
                             params['W1'].T, params['b1'].reshape(1, _D),
                             params['W2'].T, params['b2'].reshape(1, _D))
    return out[:_N]


# speedup vs baseline: 16.0759x; 16.0759x over previous
import functools

import jax
import jax.numpy as jnp
from jax import lax
from jax.experimental import pallas as pl
from jax.experimental.pallas import tpu as pltpu
from jax.experimental.pallas import tpu_sc as plsc

_N = 10000
_D = 128
_NC, _NS = 2, 16
_NW = _NC * _NS
_NPT = 320
_NPAD = _NW * _NPT
_CH = 128
_EPT_CH = 80
_EPAD = _NW * _EPT_CH * _CH
_SCHK = 8
_RPS = _NPAD // _NS

_AGG_MODES = ['sum', 'mean', 'max', 'sum', 'mean', 'max', 'sum', 'mean',
              'max', 'sum', 'max', 'mean']

_SC_PARAMS = pltpu.CompilerParams(needs_layout_passes=False)
_MESH = dict(core_axis_name="c", subcore_axis_name="s",
             num_cores=_NC, num_subcores=_NS)


def _extract(v16, k):
    return jnp.sum(jnp.where(lax.iota(jnp.int32, 16) == k, v16, 0))


def _make_stream_sum_kernel():
    @functools.partial(
        pl.kernel,
        out_type=jax.ShapeDtypeStruct((2 * _NPAD, _D), jnp.float32),
        mesh=plsc.VectorSubcoreMesh(**_MESH),
        compiler_params=_SC_PARAMS,
        scratch_types=[
            pltpu.VMEM((_EPT_CH // 2, _CH), jnp.int32),
            pltpu.VMEM((_EPT_CH // 2, _CH), jnp.int32),
            pltpu.VMEM((_CH, _D), jnp.float32),
            pltpu.VMEM((_CH, _D), jnp.float32),
            pltpu.VMEM_SHARED((_NPAD, _D), jnp.float32),
            pltpu.SemaphoreType.DMA,
            pltpu.SemaphoreType.DMA,
        ],
    )
    def k(h_hbm, src_hbm, dst_hbm, out_hbm,
          sv, dv, mv0, mv1, aggs, sem0, sem1):
        cid = lax.axis_index("c")
        sid = lax.axis_index("s")
        wid = sid * _NC + cid
        half_ch = _EPT_CH // 2

        def zrow(r, c):
            for kk in range(_D // 16):
                mv0[r, pl.ds(kk * 16, 16)] = jnp.zeros((16,), jnp.float32)
            return c
        lax.fori_loop(0, _CH, zrow, 0)
        for kk in range(_RPS // _CH):
            pltpu.sync_copy(mv0, aggs.at[pl.ds(sid * _RPS + kk * _CH, _CH)])
        plsc.subcore_barrier()

        for half in range(2):
            row0 = wid * _EPT_CH + half * half_ch
            pltpu.sync_copy(src_hbm.at[pl.ds(row0, half_ch)], sv)
            pltpu.sync_copy(dst_hbm.at[pl.ds(row0, half_ch)], dv)

            pltpu.async_copy(h_hbm.at[sv.at[0]], mv0, sem0)

            def body(i, c):
                c0 = 2 * i
                c1 = c0 + 1
                c2 = c0 + 2
                pltpu.async_copy(h_hbm.at[sv.at[c1]], mv1, sem1)
                pltpu.make_async_copy(h_hbm.at[sv.at[c0]], mv0, sem0).wait()
                pltpu.sync_copy(mv0, aggs.at[dv.at[c0]], add=True)

                @pl.when(c2 < half_ch)
                def _():
                    pltpu.async_copy(h_hbm.at[sv.at[c2]], mv0, sem0)

                pltpu.make_async_copy(h_hbm.at[sv.at[c1]], mv1, sem1).wait()
                pltpu.sync_copy(mv1, aggs.at[dv.at[c1]], add=True)
                return c
            lax.fori_loop(0, half_ch // 2, body, 0)

        plsc.subcore_barrier()
        for kk in range(_RPS // _CH):
            r = sid * _RPS + kk * _CH
            pltpu.sync_copy(aggs.at[pl.ds(r, _CH)],
                            out_hbm.at[pl.ds(cid * _NPAD + r, _CH)])

    return k


_OPS = {'sum': jnp.add, 'mean': jnp.add, 'max': jnp.maximum}


def _make_csr_kernel(op):
    ident = 0.0 if op in ('sum', 'mean') else float('-inf')

    @functools.partial(
        pl.kernel,
        out_type=jax.ShapeDtypeStruct((_NPAD * _D,), jnp.float32),
        mesh=plsc.VectorSubcoreMesh(**_MESH),
        compiler_params=_SC_PARAMS,
        scratch_types=[
            pltpu.VMEM((48,), jnp.int32),
            pltpu.VMEM((328,), jnp.int32),
            pltpu.VMEM((_SCHK, _CH), jnp.int32),
            pltpu.VMEM((16,), jnp.int32),
            pltpu.VMEM((16,), jnp.int32),
            pltpu.VMEM((_CH, _D), jnp.float32),
            pltpu.VMEM((_CH, _D), jnp.float32),
            pltpu.VMEM((_NPT * _D,), jnp.float32),
            pltpu.SemaphoreType.DMA,
            pltpu.SemaphoreType.DMA,
        ],
    )
    def k(h_hbm, src_hbm, rp_hbm, cn0_hbm, cn1_hbm, bounds_hbm, out_hbm,
          bv, rpv, sv, c0v, c1v, mv0, mv1, av, sem0, sem1):
        cid = lax.axis_index("c")
        sid = lax.axis_index("s")
        wid = sid * _NC + cid
        node_base = wid * _NPT

        def zbody(i, c):
            av[pl.ds(i * 16, 16)] = jnp.zeros((16,), jnp.float32)
            return c
        lax.fori_loop(0, _NPT * _D // 16, zbody, 0)

        pltpu.sync_copy(bounds_hbm, bv)
        pltpu.sync_copy(rp_hbm.at[pl.ds(node_base, 328)], rpv)
        b0 = bv[pl.ds(0, 16)]
        b1 = bv[pl.ds(16, 16)]
        b2 = bv[pl.ds(32, 16)]

        def bval(t):
            return _extract(b0, t) + _extract(b1, t - 16) + _extract(b2, t - 32)

        def rpval(nl):
            v16 = rpv[pl.ds(pl.multiple_of((nl // 16) * 16, 16), 16)]
            return _extract(v16, nl & 15)

        e_start = bval(wid)
        e_end = bval(wid + 1)
        sch = _SCHK * _CH
        s_lo = (e_start // sch) * sch
        n_sch = (e_end - s_lo + sch - 1) // sch

        mvs = (mv0, mv1)
        sems = (sem0, sem1)

        def sch_body(s, acc):
            row0 = pl.multiple_of((s_lo // _CH) + s * _SCHK, _SCHK)
            pltpu.sync_copy(src_hbm.at[pl.ds(row0, _SCHK)], sv)
            pltpu.sync_copy(cn0_hbm.at[pl.ds(row0, 16)], c0v)
            pltpu.sync_copy(cn1_hbm.at[pl.ds(row0, 16)], c1v)
            cn0_16 = c0v[...]
            cn1_16 = c1v[...]
            pltpu.async_copy(h_hbm.at[sv.at[0]], mv0, sem0)
            for kk in range(_SCHK):
                mk, sk = mvs[kk % 2], sems[kk % 2]
                if kk + 1 < _SCHK:
                    pltpu.async_copy(h_hbm.at[sv.at[kk + 1]],
                                     mvs[(kk + 1) % 2], sems[(kk + 1) % 2])
                pltpu.make_async_copy(h_hbm.at[sv.at[kk]], mk, sk).wait()
                ch_lo = s_lo + s * sch + kk * _CH
                ch_hi = ch_lo + _CH
                nlo = jnp.maximum(_extract(cn0_16, kk), node_base)
                nhi = jnp.minimum(_extract(cn1_16, kk), node_base + _NPT - 1)

                def node_body(n, acc2, _mk=mk, _lo=ch_lo, _hi=ch_hi):
                    nl = n - node_base
                    rpn = rpval(nl)
                    rpn1 = rpval(nl + 1)
                    e0 = jnp.maximum(rpn, _lo)
                    e1 = jnp.minimum(rpn1, _hi)

                    def edge_body(e, acc3):
                        el = e - _lo
                        return tuple(
                            _OPS[op](acc3[cc], _mk[el, pl.ds(cc * 16, 16)])
                            for cc in range(8))
                    acc2 = lax.fori_loop(e0, e1, edge_body, acc2)
                    done = jnp.logical_and(rpn1 <= _hi, rpn1 > rpn)

                    @pl.when(done)
                    def _():
                        if op == 'mean':
                            degv = jnp.full((16,), rpn1 - rpn,
                                            jnp.int32).astype(jnp.float32)
                            sc = 1.0 / degv
                        else:
                            sc = 1.0
                        for cc in range(8):
                            av[pl.ds(nl * _D + cc * 16, 16)] = acc2[cc] * sc
                    return tuple(
                        jnp.where(done, ident, acc2[cc]).astype(jnp.float32)
                        for cc in range(8))
                acc = lax.fori_loop(nlo, nhi + 1, node_body, acc)
            return acc

        acc0 = tuple(jnp.full((16,), ident, jnp.float32) for _ in range(8))
        lax.fori_loop(0, n_sch, sch_body, acc0)
        pltpu.sync_copy(av.at[pl.ds(0, _NPT * _D)],
                        out_hbm.at[pl.ds(node_base * _D, _NPT * _D)])

    return k


_agg_stream_sum = _make_stream_sum_kernel()
_agg_csr = {m: _make_csr_kernel(m) for m in ('sum', 'mean', 'max')}

_BR = 1024


def _sage2_body(p0_ref, p1_ref, sc_ref, h_ref, wl_ref, b_ref, wr_ref, o_ref):
    a = (p0_ref[...] + p1_ref[...]) * sc_ref[...]
    y = jnp.dot(a, wl_ref[...], preferred_element_type=jnp.float32)
    y = y + jnp.dot(h_ref[...], wr_ref[...], preferred_element_type=jnp.float32)
    o_ref[...] = jnp.maximum(y + b_ref[...], 0.0)


def _sage1_body(agg_ref, h_ref, wl_ref, b_ref, wr_ref, o_ref):
    y = jnp.dot(agg_ref[...], wl_ref[...], preferred_element_type=jnp.float32)
    y = y + jnp.dot(h_ref[...], wr_ref[...], preferred_element_type=jnp.float32)
    o_ref[...] = jnp.maximum(y + b_ref[...], 0.0)


def _final_body(p0_ref, p1_ref, sc_ref, h_ref, wl_ref, b_ref, wr_ref,
                w1_ref, b1_ref, w2_ref, b2_ref, o_ref):
    a = (p0_ref[...] + p1_ref[...]) * sc_ref[...]
    y = jnp.dot(a, wl_ref[...], preferred_element_type=jnp.float32)
    y = y + jnp.dot(h_ref[...], wr_ref[...], preferred_element_type=jnp.float32)
    y = jnp.maximum(y + b_ref[...], 0.0)
    y = jnp.maximum(
        jnp.dot(y, w1_ref[...], preferred_element_type=jnp.float32)
        + b1_ref[...], 0.0)
    o_ref[...] = (jnp.dot(y, w2_ref[...], preferred_element_type=jnp.float32)
                  + b2_ref[...])


def _final1_body(agg_ref, h_ref, wl_ref, b_ref, wr_ref,
                 w1_ref, b1_ref, w2_ref, b2_ref, o_ref):
    y = jnp.dot(agg_ref[...], wl_ref[...], preferred_element_type=jnp.float32)
    y = y + jnp.dot(h_ref[...], wr_ref[...], preferred_element_type=jnp.float32)
    y = jnp.maximum(y + b_ref[...], 0.0)
    y = jnp.maximum(
        jnp.dot(y, w1_ref[...], preferred_element_type=jnp.float32)
        + b1_ref[...], 0.0)
    o_ref[...] = (jnp.dot(y, w2_ref[...], preferred_element_type=jnp.float32)
                  + b2_ref[...])


def _row_spec():
    return pl.BlockSpec((_BR, _D), lambda i: (i, 0))


def _scale_spec():
    return pl.BlockSpec((_BR, 1), lambda i: (i, 0))


def _full_spec(shape):
    return pl.BlockSpec(shape, lambda i: (0, 0))


def _tc_call(body, specs, args):
    return pl.pallas_call(
        body,
        grid=(_NPAD // _BR,),
        in_specs=specs,
        out_specs=_row_spec(),
        out_shape=jax.ShapeDtypeStruct((_NPAD, _D), jnp.float32),
    )(*args)


def _tc_sage2(p0, p1, scale, h, wlT, bl, wrT):
    specs = [_row_spec(), _row_spec(), _scale_spec(), _row_spec(),
             _full_spec((_D, _D)), _full_spec((1, _D)), _full_spec((_D, _D))]
    return _tc_call(_sage2_body, specs, (p0, p1, scale, h, wlT, bl, wrT))


def _tc_sage1(agg, h, wlT, bl, wrT):
    specs = [_row_spec(), _row_spec(), _full_spec((_D, _D)),
             _full_spec((1, _D)), _full_spec((_D, _D))]
    return _tc_call(_sage1_body, specs, (agg, h, wlT, bl, wrT))


def _tc_final1(agg, h, wlT, bl, wrT, w1T, b1, w2T, b2):
    specs = [_row_spec(), _row_spec(), _full_spec((_D, _D)),
             _full_spec((1, _D)), _full_spec((_D, _D)), _full_spec((_D, _D)),
             _full_spec((1, _D)), _full_spec((_D, _D)), _full_spec((1, _D))]
    return _tc_call(_final1_body, specs, (agg, h, wlT, bl, wrT, w1T, b1, w2T, b2))


def _tc_final(p0, p1, scale, h, wlT, bl, wrT, w1T, b1, w2T, b2):
    specs = [_row_spec(), _row_spec(), _scale_spec(), _row_spec(),
             _full_spec((_D, _D)), _full_spec((1, _D)), _full_spec((_D, _D)),
             _full_spec((_D, _D)), _full_spec((1, _D)),
             _full_spec((_D, _D)), _full_spec((1, _D))]
    return _tc_call(_final_body, specs,
                    (p0, p1, scale, h, wlT, bl, wrT, w1T, b1, w2T, b2))


def kernel(x, edge_index, params):
    src = edge_index[0].astype(jnp.int32)
    dst = edge_index[1].astype(jnp.int32)
    n_e = src.shape[0]
    n_rows = n_e // _CH

    dst_s, src_s = lax.sort((dst, src), num_keys=1)
    src_s2 = jnp.concatenate(
        [src_s.reshape(n_rows, _CH),
         jnp.zeros((4, _CH), jnp.int32)])
    dst_s2 = dst_s.reshape(n_rows, _CH)
    cn_pad = jnp.full((12,), _NPAD - 1, jnp.int32)
    cn0 = jnp.concatenate([dst_s2[:, 0], cn_pad])
    cn1 = jnp.concatenate([dst_s2[:, -1], cn_pad])

    cnt = jnp.zeros((_NPAD + 7,), jnp.int32).at[dst].add(
        1, mode='drop', unique_indices=False)
    rowptr = jnp.concatenate(
        [jnp.zeros((1,), jnp.int32), jnp.cumsum(cnt)])
    rp_pad = rowptr
    tile_nodes = jnp.arange(0, _NPAD + 1, _NPT, dtype=jnp.int32)
    bounds = rowptr[tile_nodes]
    bounds_pad = jnp.zeros((48,), jnp.int32).at[:_NW + 1].set(bounds)

    h = jnp.zeros((_NPAD, _D), jnp.float32).at[:_N].set(x)

    for i, aggr in enumerate(_AGG_MODES):
        wlT = params['Wl%d' % i].T
        bl = params['bl%d' % i].reshape(1, _D)
        wrT = params['Wr%d' % i].T
        agg = _agg_csr[aggr](h, src_s2, rp_pad, cn0, cn1,
                             bounds_pad).reshape(_NPAD, _D)
        if i < len(_AGG_MODES) - 1:
            h = _tc_sage1(agg, h, wlT, bl, wrT)
        else:
            out = _tc_final1(agg, h, wlT, bl, wrT,
                             params['W1'].T, params['b1'].reshape(1, _D),
                             params['W2'].T, params['b2'].reshape(1, _D))
    return out[:_N]


# speedup vs baseline: 16.0907x; 1.0009x over previous
import functools

import jax
import jax.numpy as jnp
from jax import lax
from jax.experimental import pallas as pl
from jax.experimental.pallas import tpu as pltpu
from jax.experimental.pallas import tpu_sc as plsc

_N = 10000
_D = 128
_NC, _NS = 2, 16
_NW = _NC * _NS
_NPT = 320
_NPAD = _NW * _NPT
_CH = 128
_SCHK = 8

_AGG_MODES = ['sum', 'mean', 'max', 'sum', 'mean', 'max', 'sum', 'mean',
              'max', 'sum', 'max', 'mean']

_SC_PARAMS = pltpu.CompilerParams(needs_layout_passes=False)
_MESH = dict(core_axis_name="c", subcore_axis_name="s",
             num_cores=_NC, num_subcores=_NS)


def _extract(v16, k):
    return jnp.sum(jnp.where(lax.iota(jnp.int32, 16) == k, v16, 0))


_OPS = {'sum': jnp.add, 'mean': jnp.add, 'max': jnp.maximum}


def _make_csr_kernel(op):
    ident = 0.0 if op in ('sum', 'mean') else float('-inf')

    @functools.partial(
        pl.kernel,
        out_type=jax.ShapeDtypeStruct((_NPAD * _D,), jnp.float32),
        mesh=plsc.VectorSubcoreMesh(**_MESH),
        compiler_params=_SC_PARAMS,
        scratch_types=[
            pltpu.VMEM((48,), jnp.int32),
            pltpu.VMEM((328,), jnp.int32),
            pltpu.VMEM((_SCHK, _CH), jnp.int32),
            pltpu.VMEM((16,), jnp.int32),
            pltpu.VMEM((16,), jnp.int32),
            pltpu.VMEM((_CH, _D), jnp.float32),
            pltpu.VMEM((_CH, _D), jnp.float32),
            pltpu.VMEM((_NPT * _D,), jnp.float32),
            pltpu.SemaphoreType.DMA,
            pltpu.SemaphoreType.DMA,
        ],
    )
    def k(h_hbm, src_hbm, rp_hbm, cn0_hbm, cn1_hbm, bounds_hbm, out_hbm,
          bv, rpv, sv, c0v, c1v, mv0, mv1, av, sem0, sem1):
        cid = lax.axis_index("c")
        sid = lax.axis_index("s")
        wid = sid * _NC + cid
        node_base = wid * _NPT

        def zbody(i, c):
            av[pl.ds(i * 16, 16)] = jnp.zeros((16,), jnp.float32)
            return c
        lax.fori_loop(0, _NPT * _D // 16, zbody, 0)

        pltpu.sync_copy(bounds_hbm, bv)
        pltpu.sync_copy(rp_hbm.at[pl.ds(node_base, 328)], rpv)
        b0 = bv[pl.ds(0, 16)]
        b1 = bv[pl.ds(16, 16)]
        b2 = bv[pl.ds(32, 16)]

        def bval(t):
            return _extract(b0, t) + _extract(b1, t - 16) + _extract(b2, t - 32)

        def rpval(nl):
            v16 = rpv[pl.ds(pl.multiple_of((nl // 16) * 16, 16), 16)]
            return _extract(v16, nl & 15)

        e_start = bval(wid)
        e_end = bval(wid + 1)
        sch = _SCHK * _CH
        s_lo = (e_start // sch) * sch
        n_sch = (e_end - s_lo + sch - 1) // sch

        mvs = (mv0, mv1)
        sems = (sem0, sem1)

        def sch_body(s, acc):
            row0 = pl.multiple_of((s_lo // _CH) + s * _SCHK, _SCHK)
            pltpu.sync_copy(src_hbm.at[pl.ds(row0, _SCHK)], sv)
            pltpu.sync_copy(cn0_hbm.at[pl.ds(row0, 16)], c0v)
            pltpu.sync_copy(cn1_hbm.at[pl.ds(row0, 16)], c1v)
            cn0_16 = c0v[...]
            cn1_16 = c1v[...]
            pltpu.async_copy(h_hbm.at[sv.at[0]], mv0, sem0)
            for kk in range(_SCHK):
                mk, sk = mvs[kk % 2], sems[kk % 2]
                if kk + 1 < _SCHK:
                    pltpu.async_copy(h_hbm.at[sv.at[kk + 1]],
                                     mvs[(kk + 1) % 2], sems[(kk + 1) % 2])
                pltpu.make_async_copy(h_hbm.at[sv.at[kk]], mk, sk).wait()
                ch_lo = s_lo + s * sch + kk * _CH
                ch_hi = ch_lo + _CH
                nlo = jnp.maximum(_extract(cn0_16, kk), node_base)
                nhi = jnp.minimum(_extract(cn1_16, kk), node_base + _NPT - 1)

                def node_body(n, acc2, _mk=mk, _lo=ch_lo, _hi=ch_hi):
                    nl = n - node_base
                    rpn = rpval(nl)
                    rpn1 = rpval(nl + 1)
                    e0 = jnp.maximum(rpn, _lo)
                    e1 = jnp.minimum(rpn1, _hi)

                    def edge_body(e, acc3):
                        el = e - _lo
                        return tuple(
                            _OPS[op](acc3[cc], _mk[el, pl.ds(cc * 16, 16)])
                            for cc in range(8))
                    acc2 = lax.fori_loop(e0, e1, edge_body, acc2)
                    done = jnp.logical_and(rpn1 <= _hi, rpn1 > rpn)

                    @pl.when(done)
                    def _():
                        if op == 'mean':
                            degv = jnp.full((16,), rpn1 - rpn,
                                            jnp.int32).astype(jnp.float32)
                            sc = 1.0 / degv
                        else:
                            sc = 1.0
                        for cc in range(8):
                            av[pl.ds(nl * _D + cc * 16, 16)] = acc2[cc] * sc
                    return tuple(
                        jnp.where(done, ident, acc2[cc]).astype(jnp.float32)
                        for cc in range(8))
                acc = lax.fori_loop(nlo, nhi + 1, node_body, acc)
            return acc

        acc0 = tuple(jnp.full((16,), ident, jnp.float32) for _ in range(8))
        lax.fori_loop(0, n_sch, sch_body, acc0)
        pltpu.sync_copy(av.at[pl.ds(0, _NPT * _D)],
                        out_hbm.at[pl.ds(node_base * _D, _NPT * _D)])

    return k


_agg_csr = {m: _make_csr_kernel(m) for m in ('sum', 'mean', 'max')}

_BR = 1024


def _sage1_body(agg_ref, h_ref, wl_ref, b_ref, wr_ref, o_ref):
    y = jnp.dot(agg_ref[...], wl_ref[...], preferred_element_type=jnp.float32)
    y = y + jnp.dot(h_ref[...], wr_ref[...], preferred_element_type=jnp.float32)
    o_ref[...] = jnp.maximum(y + b_ref[...], 0.0)


def _final1_body(agg_ref, h_ref, wl_ref, b_ref, wr_ref,
                 w1_ref, b1_ref, w2_ref, b2_ref, o_ref):
    y = jnp.dot(agg_ref[...], wl_ref[...], preferred_element_type=jnp.float32)
    y = y + jnp.dot(h_ref[...], wr_ref[...], preferred_element_type=jnp.float32)
    y = jnp.maximum(y + b_ref[...], 0.0)
    y = jnp.maximum(
        jnp.dot(y, w1_ref[...], preferred_element_type=jnp.float32)
        + b1_ref[...], 0.0)
    o_ref[...] = (jnp.dot(y, w2_ref[...], preferred_element_type=jnp.float32)
                  + b2_ref[...])


def _row_spec():
    return pl.BlockSpec((_BR, _D), lambda i: (i, 0))


def _full_spec(shape):
    return pl.BlockSpec(shape, lambda i: (0, 0))


def _tc_call(body, specs, args):
    return pl.pallas_call(
        body,
        grid=(_NPAD // _BR,),
        in_specs=specs,
        out_specs=_row_spec(),
        out_shape=jax.ShapeDtypeStruct((_NPAD, _D), jnp.float32),
    )(*args)


def _tc_sage1(agg, h, wlT, bl, wrT):
    specs = [_row_spec(), _row_spec(), _full_spec((_D, _D)),
             _full_spec((1, _D)), _full_spec((_D, _D))]
    return _tc_call(_sage1_body, specs, (agg, h, wlT, bl, wrT))


def _tc_final1(agg, h, wlT, bl, wrT, w1T, b1, w2T, b2):
    specs = [_row_spec(), _row_spec(), _full_spec((_D, _D)),
             _full_spec((1, _D)), _full_spec((_D, _D)), _full_spec((_D, _D)),
             _full_spec((1, _D)), _full_spec((_D, _D)), _full_spec((1, _D))]
    return _tc_call(_final1_body, specs, (agg, h, wlT, bl, wrT, w1T, b1, w2T, b2))


def kernel(x, edge_index, params):
    src = edge_index[0].astype(jnp.int32)
    dst = edge_index[1].astype(jnp.int32)
    n_e = src.shape[0]
    n_rows = n_e // _CH

    dst_s, src_s = lax.sort((dst, src), num_keys=1)
    src_s2 = jnp.concatenate(
        [src_s.reshape(n_rows, _CH),
         jnp.zeros((4, _CH), jnp.int32)])
    dst_s2 = dst_s.reshape(n_rows, _CH)
    cn_pad = jnp.full((12,), _NPAD - 1, jnp.int32)
    cn0 = jnp.concatenate([dst_s2[:, 0], cn_pad])
    cn1 = jnp.concatenate([dst_s2[:, -1], cn_pad])

    cnt = jnp.zeros((_NPAD + 7,), jnp.int32).at[dst].add(
        1, mode='drop', unique_indices=False)
    rowptr = jnp.concatenate(
        [jnp.zeros((1,), jnp.int32), jnp.cumsum(cnt)])
    rp_pad = rowptr
    tile_nodes = jnp.arange(0, _NPAD + 1, _NPT, dtype=jnp.int32)
    bounds = rowptr[tile_nodes]
    bounds_pad = jnp.zeros((48,), jnp.int32).at[:_NW + 1].set(bounds)

    h = jnp.zeros((_NPAD, _D), jnp.float32).at[:_N].set(x)

    for i, aggr in enumerate(_AGG_MODES):
        wlT = params['Wl%d' % i].T
        bl = params['bl%d' % i].reshape(1, _D)
        wrT = params['Wr%d' % i].T
        agg = _agg_csr[aggr](h, src_s2, rp_pad, cn0, cn1,
                             bounds_pad).reshape(_NPAD, _D)
        if i < len(_AGG_MODES) - 1:
            h = _tc_sage1(agg, h, wlT, bl, wrT)
        else:
            out = _tc_final1(agg, h, wlT, bl, wrT,
                             params['W1'].T, params['b1'].reshape(1, _D),
                             params['W2'].T, params['b2'].reshape(1, _D))
    return out[:_N]
